# Initial kernel scaffold; baseline (speedup 1.0000x reference)
import jax
import jax.numpy as jnp
from jax.experimental import pallas as pl


def kernel(x, edge_index_shards, W1, b1, W2, b2, W3, b3):
    raise NotImplementedError("write your pallas kernel here")


# speedup vs baseline: 20.2922x; 20.2922x over previous
import jax
import jax.numpy as jnp
from jax import lax
from jax.experimental import pallas as pl
from jax.experimental.pallas import tpu as pltpu
from jax.experimental.pallas import tpu_sc as plsc

N = 10000
D = 128
HD = D // 2
E = 320000
NC = 2
NS = 16
NW = NC * NS
CHUNK = 128
NBUF = 4
CPT = 160
DCPT = 80
EP = NS * CPT * CHUNK
NGRP = CPT // NBUF
DNBUF = 8
DNGRP = DCPT // DNBUF
ACC_ROWS = 10112
ZROWS = ACC_ROWS // NS
DEG_W = 16
BLK = 1000
GRID = N // BLK

_mesh = plsc.VectorSubcoreMesh(core_axis_name="c", subcore_axis_name="s",
                               num_cores=NC, num_subcores=NS)


def _deg_body(dst_hbm, ones_hbm, zeros_hbm, out_hbm, idx_d, onesbuf, acc, sem):
  c = lax.axis_index("c")
  s = lax.axis_index("s")
  w = c * NS + s
  pltpu.sync_copy(dst_hbm.at[w], idx_d)
  pltpu.sync_copy(ones_hbm, onesbuf)
  pltpu.sync_copy(zeros_hbm, acc.at[pl.ds(s * ZROWS, ZROWS)])
  plsc.subcore_barrier()

  @pl.loop(0, DNGRP)
  def _grp(gi):
    base = gi * DNBUF
    descs = [
        pltpu.async_copy(onesbuf, acc.at[idx_d.at[base + b]], sem, add=True)
        for b in range(DNBUF)
    ]
    for d in descs:
      d.wait()

  plsc.subcore_barrier()
  pltpu.sync_copy(acc.at[pl.ds(s * ZROWS, ZROWS)],
                  out_hbm.at[c, pl.ds(s * ZROWS, ZROWS)])


_deg = pl.kernel(
    _deg_body,
    out_type=jax.ShapeDtypeStruct((NC, ACC_ROWS, DEG_W), jnp.float32),
    mesh=_mesh,
    scratch_types=[
        pltpu.VMEM((DCPT, CHUNK), jnp.int32),
        pltpu.VMEM((CHUNK, DEG_W), jnp.float32),
        pltpu.VMEM_SHARED((ACC_ROWS, DEG_W), jnp.float32),
        pltpu.SemaphoreType.DMA,
    ],
)


def _agg_body(g_hbm, src_hbm, dst_hbm, zeros_hbm, out_hbm,
              idx_s, idx_d, bufs, acc, gsem, ssem):
  c = lax.axis_index("c")
  s = lax.axis_index("s")
  pltpu.sync_copy(src_hbm.at[s], idx_s)
  pltpu.sync_copy(dst_hbm.at[s], idx_d)
  pltpu.sync_copy(zeros_hbm, acc.at[pl.ds(s * ZROWS, ZROWS)])
  plsc.subcore_barrier()

  @pl.loop(0, NGRP)
  def _grp(gi):
    base = gi * NBUF
    gds = []
    for b in range(NBUF):
      gds.append(
          pltpu.async_copy(g_hbm.at[c].at[idx_s.at[base + b]], bufs.at[b],
                           gsem.at[b]))
    sds = []
    for b in range(NBUF):
      gds[b].wait()
      sds.append(
          pltpu.async_copy(bufs.at[b], acc.at[idx_d.at[base + b]],
                           ssem.at[b], add=True))
    for d in sds:
      d.wait()

  plsc.subcore_barrier()
  pltpu.sync_copy(acc.at[pl.ds(s * ZROWS, ZROWS)],
                  out_hbm.at[c, pl.ds(s * ZROWS, ZROWS)])


_agg = pl.kernel(
    _agg_body,
    out_type=jax.ShapeDtypeStruct((NC, ACC_ROWS, HD), jnp.float32),
    mesh=_mesh,
    compiler_params=pltpu.CompilerParams(use_tc_tiling_on_sc=False),
    scratch_types=[
        pltpu.VMEM((CPT, CHUNK), jnp.int32),
        pltpu.VMEM((CPT, CHUNK), jnp.int32),
        pltpu.VMEM((NBUF, CHUNK, HD), jnp.float32),
        pltpu.VMEM_SHARED((ACC_ROWS, HD), jnp.float32),
        pltpu.SemaphoreType.DMA((NBUF,)),
        pltpu.SemaphoreType.DMA((NBUF,)),
    ],
)


def _d1_body(x_ref, w_ref, degp_ref, g_ref, dis_ref):
  deg = 1.0 + degp_ref[0, :, 0:1] + degp_ref[1, :, 0:1]
  dis = lax.rsqrt(deg)
  g = jnp.dot(x_ref[...], w_ref[...], preferred_element_type=jnp.float32) * dis
  g_ref[0] = g[:, :HD]
  g_ref[1] = g[:, HD:]
  dis_ref[...] = dis


_d1 = pl.pallas_call(
    _d1_body,
    grid=(GRID,),
    in_specs=[
        pl.BlockSpec((BLK, D), lambda i: (i, 0)),
        pl.BlockSpec((D, D), lambda i: (0, 0)),
        pl.BlockSpec((NC, BLK, DEG_W), lambda i: (0, i, 0)),
    ],
    out_specs=[
        pl.BlockSpec((NC, BLK, HD), lambda i: (0, i, 0)),
        pl.BlockSpec((BLK, 1), lambda i: (i, 0)),
    ],
    out_shape=[
        jax.ShapeDtypeStruct((NC, N, HD), jnp.float32),
        jax.ShapeDtypeStruct((N, 1), jnp.float32),
    ],
)


def _dmid_body(p_ref, g_ref, dis_ref, b_ref, w_ref, gn_ref):
  dis = dis_ref[...]
  agg = jnp.concatenate([p_ref[0] + g_ref[0], p_ref[1] + g_ref[1]], axis=1)
  z = jnp.maximum(dis * agg + b_ref[...], 0.0)
  gn = jnp.dot(z, w_ref[...], preferred_element_type=jnp.float32) * dis
  gn_ref[0] = gn[:, :HD]
  gn_ref[1] = gn[:, HD:]


_dmid = pl.pallas_call(
    _dmid_body,
    grid=(GRID,),
    in_specs=[
        pl.BlockSpec((NC, BLK, HD), lambda i: (0, i, 0)),
        pl.BlockSpec((NC, BLK, HD), lambda i: (0, i, 0)),
        pl.BlockSpec((BLK, 1), lambda i: (i, 0)),
        pl.BlockSpec((1, D), lambda i: (0, 0)),
        pl.BlockSpec((D, D), lambda i: (0, 0)),
    ],
    out_specs=pl.BlockSpec((NC, BLK, HD), lambda i: (0, i, 0)),
    out_shape=jax.ShapeDtypeStruct((NC, N, HD), jnp.float32),
)


def _dfin_body(p_ref, g_ref, dis_ref, b_ref, out_ref):
  agg = jnp.concatenate([p_ref[0] + g_ref[0], p_ref[1] + g_ref[1]], axis=1)
  out_ref[...] = dis_ref[...] * agg + b_ref[...]


_dfin = pl.pallas_call(
    _dfin_body,
    grid=(GRID,),
    in_specs=[
        pl.BlockSpec((NC, BLK, HD), lambda i: (0, i, 0)),
        pl.BlockSpec((NC, BLK, HD), lambda i: (0, i, 0)),
        pl.BlockSpec((BLK, 1), lambda i: (i, 0)),
        pl.BlockSpec((1, D), lambda i: (0, 0)),
    ],
    out_specs=pl.BlockSpec((BLK, D), lambda i: (i, 0)),
    out_shape=jax.ShapeDtypeStruct((N, D), jnp.float32),
)


def kernel(x, edge_index_shards, W1, b1, W2, b2, W3, b3):
  ei = edge_index_shards.astype(jnp.int32)
  pad = jnp.arange(EP - E, dtype=jnp.int32)
  pad_src = (pad * 97) % N
  pad_dst = N + (pad % (ACC_ROWS - N))
  src = jnp.concatenate([ei[0], pad_src])
  dst = jnp.concatenate([ei[1], pad_dst])
  src_a = src.reshape(NS, CPT, CHUNK)
  dst_a = dst.reshape(NS, CPT, CHUNK)
  dst_d = dst.reshape(NW, DCPT, CHUNK)

  zeros_h = jnp.zeros((ZROWS, HD), jnp.float32)
  zeros_w = jnp.zeros((ZROWS, DEG_W), jnp.float32)
  ones_w = jnp.ones((CHUNK, DEG_W), jnp.float32)

  degp = _deg(dst_d, ones_w, zeros_w)
  g1, dis = _d1(x, W1, degp)
  p1 = _agg(g1, src_a, dst_a, zeros_h)
  g2 = _dmid(p1, g1, dis, b1.reshape(1, D), W2)
  p2 = _agg(g2, src_a, dst_a, zeros_h)
  g3 = _dmid(p2, g2, dis, b2.reshape(1, D), W3)
  p3 = _agg(g3, src_a, dst_a, zeros_h)
  return _dfin(p3, g3, dis, b3.reshape(1, D))


# speedup vs baseline: 21.5859x; 1.0638x over previous
import jax
import jax.numpy as jnp
from jax import lax
from jax.experimental import pallas as pl
from jax.experimental.pallas import tpu as pltpu
from jax.experimental.pallas import tpu_sc as plsc

N = 10000
D = 128
HD = D // 2
E = 320000
NC = 2
NS = 16
NW = NC * NS
CHUNK = 128
NBUF = 6
CPT = 160
NGRP = CPT // NBUF
TAIL = CPT % NBUF
DCPT = 80
EP = NS * CPT * CHUNK
DNBUF = 8
DNGRP = DCPT // DNBUF
ACC_ROWS = 10112
ZROWS = ACC_ROWS // NS
DEG_W = 16
BLK = 1000
GRID = N // BLK

_mesh = plsc.VectorSubcoreMesh(core_axis_name="c", subcore_axis_name="s",
                               num_cores=NC, num_subcores=NS)


def _deg_body(dst_hbm, ones_hbm, zeros_hbm, out_hbm, idx_d, onesbuf, acc, sem):
  c = lax.axis_index("c")
  s = lax.axis_index("s")
  w = c * NS + s
  pltpu.sync_copy(dst_hbm.at[w], idx_d)
  pltpu.sync_copy(ones_hbm, onesbuf)
  pltpu.sync_copy(zeros_hbm, acc.at[pl.ds(s * ZROWS, ZROWS)])
  plsc.subcore_barrier()

  @pl.loop(0, DNGRP)
  def _grp(gi):
    base = gi * DNBUF
    descs = [
        pltpu.async_copy(onesbuf, acc.at[idx_d.at[base + b]], sem, add=True)
        for b in range(DNBUF)
    ]
    for d in descs:
      d.wait()

  plsc.subcore_barrier()
  pltpu.sync_copy(acc.at[pl.ds(s * ZROWS, ZROWS)],
                  out_hbm.at[c, pl.ds(s * ZROWS, ZROWS)])


_deg = pl.kernel(
    _deg_body,
    out_type=jax.ShapeDtypeStruct((NC, ACC_ROWS, DEG_W), jnp.float32),
    mesh=_mesh,
    compiler_params=pltpu.CompilerParams(use_tc_tiling_on_sc=False),
    scratch_types=[
        pltpu.VMEM((DCPT, CHUNK), jnp.int32),
        pltpu.VMEM((CHUNK, DEG_W), jnp.float32),
        pltpu.VMEM_SHARED((ACC_ROWS, DEG_W), jnp.float32),
        pltpu.SemaphoreType.DMA,
    ],
)


def _agg_body(g_hbm, src_hbm, dst_hbm, zeros_hbm, out_hbm,
              idx_s, idx_d, bufs, acc, gsem, ssem):
  c = lax.axis_index("c")
  s = lax.axis_index("s")
  pltpu.sync_copy(src_hbm.at[s], idx_s)
  pltpu.sync_copy(dst_hbm.at[s], idx_d)
  pltpu.sync_copy(zeros_hbm, acc.at[pl.ds(s * ZROWS, ZROWS)])
  plsc.subcore_barrier()

  def _run_group(base, nb):
    gds = []
    for b in range(nb):
      gds.append(
          pltpu.async_copy(g_hbm.at[c].at[idx_s.at[base + b]], bufs.at[b],
                           gsem.at[b]))
    sds = []
    for b in range(nb):
      gds[b].wait()
      sds.append(
          pltpu.async_copy(bufs.at[b], acc.at[idx_d.at[base + b]],
                           ssem.at[b], add=True))
    for d in sds:
      d.wait()

  @pl.loop(0, NGRP)
  def _grp(gi):
    _run_group(gi * NBUF, NBUF)

  if TAIL:
    _run_group(NGRP * NBUF, TAIL)

  plsc.subcore_barrier()
  pltpu.sync_copy(acc.at[pl.ds(s * ZROWS, ZROWS)],
                  out_hbm.at[c, pl.ds(s * ZROWS, ZROWS)])


_agg = pl.kernel(
    _agg_body,
    out_type=jax.ShapeDtypeStruct((NC, ACC_ROWS, HD), jnp.float32),
    mesh=_mesh,
    compiler_params=pltpu.CompilerParams(use_tc_tiling_on_sc=False),
    scratch_types=[
        pltpu.VMEM((CPT, CHUNK), jnp.int32),
        pltpu.VMEM((CPT, CHUNK), jnp.int32),
        pltpu.VMEM((NBUF, CHUNK, HD), jnp.float32),
        pltpu.VMEM_SHARED((ACC_ROWS, HD), jnp.float32),
        pltpu.SemaphoreType.DMA((NBUF,)),
        pltpu.SemaphoreType.DMA((NBUF,)),
    ],
)


def _d1_body(x_ref, w_ref, degp_ref, g_ref, dis_ref):
  deg = 1.0 + degp_ref[0, :, 0:1] + degp_ref[1, :, 0:1]
  dis = lax.rsqrt(deg)
  g = jnp.dot(x_ref[...], w_ref[...], preferred_element_type=jnp.float32) * dis
  g_ref[0] = g[:, :HD]
  g_ref[1] = g[:, HD:]
  dis_ref[...] = dis


_d1 = pl.pallas_call(
    _d1_body,
    grid=(GRID,),
    in_specs=[
        pl.BlockSpec((BLK, D), lambda i: (i, 0)),
        pl.BlockSpec((D, D), lambda i: (0, 0)),
        pl.BlockSpec((NC, BLK, DEG_W), lambda i: (0, i, 0)),
    ],
    out_specs=[
        pl.BlockSpec((NC, BLK, HD), lambda i: (0, i, 0)),
        pl.BlockSpec((BLK, 1), lambda i: (i, 0)),
    ],
    out_shape=[
        jax.ShapeDtypeStruct((NC, N, HD), jnp.float32),
        jax.ShapeDtypeStruct((N, 1), jnp.float32),
    ],
)


def _dmid_body(p_ref, g_ref, dis_ref, b_ref, w_ref, gn_ref):
  dis = dis_ref[...]
  agg = jnp.concatenate([p_ref[0] + g_ref[0], p_ref[1] + g_ref[1]], axis=1)
  z = jnp.maximum(dis * agg + b_ref[...], 0.0)
  gn = jnp.dot(z, w_ref[...], preferred_element_type=jnp.float32) * dis
  gn_ref[0] = gn[:, :HD]
  gn_ref[1] = gn[:, HD:]


_dmid = pl.pallas_call(
    _dmid_body,
    grid=(GRID,),
    in_specs=[
        pl.BlockSpec((NC, BLK, HD), lambda i: (0, i, 0)),
        pl.BlockSpec((NC, BLK, HD), lambda i: (0, i, 0)),
        pl.BlockSpec((BLK, 1), lambda i: (i, 0)),
        pl.BlockSpec((1, D), lambda i: (0, 0)),
        pl.BlockSpec((D, D), lambda i: (0, 0)),
    ],
    out_specs=pl.BlockSpec((NC, BLK, HD), lambda i: (0, i, 0)),
    out_shape=jax.ShapeDtypeStruct((NC, N, HD), jnp.float32),
)


def _dfin_body(p_ref, g_ref, dis_ref, b_ref, out_ref):
  agg = jnp.concatenate([p_ref[0] + g_ref[0], p_ref[1] + g_ref[1]], axis=1)
  out_ref[...] = dis_ref[...] * agg + b_ref[...]


_dfin = pl.pallas_call(
    _dfin_body,
    grid=(GRID,),
    in_specs=[
        pl.BlockSpec((NC, BLK, HD), lambda i: (0, i, 0)),
        pl.BlockSpec((NC, BLK, HD), lambda i: (0, i, 0)),
        pl.BlockSpec((BLK, 1), lambda i: (i, 0)),
        pl.BlockSpec((1, D), lambda i: (0, 0)),
    ],
    out_specs=pl.BlockSpec((BLK, D), lambda i: (i, 0)),
    out_shape=jax.ShapeDtypeStruct((N, D), jnp.float32),
)


def kernel(x, edge_index_shards, W1, b1, W2, b2, W3, b3):
  ei = edge_index_shards.astype(jnp.int32)
  pad = jnp.arange(EP - E, dtype=jnp.int32)
  pad_src = (pad * 97) % N
  pad_dst = N + (pad % (ACC_ROWS - N))
  src = jnp.concatenate([ei[0], pad_src])
  dst = jnp.concatenate([ei[1], pad_dst])
  src_a = src.reshape(NS, CPT, CHUNK)
  dst_a = dst.reshape(NS, CPT, CHUNK)
  dst_d = dst.reshape(NW, DCPT, CHUNK)

  zeros_h = jnp.zeros((ZROWS, HD), jnp.float32)
  zeros_w = jnp.zeros((ZROWS, DEG_W), jnp.float32)
  ones_w = jnp.ones((CHUNK, DEG_W), jnp.float32)

  degp = _deg(dst_d, ones_w, zeros_w)
  g1, dis = _d1(x, W1, degp)
  p1 = _agg(g1, src_a, dst_a, zeros_h)
  g2 = _dmid(p1, g1, dis, b1.reshape(1, D), W2)
  p2 = _agg(g2, src_a, dst_a, zeros_h)
  g3 = _dmid(p2, g2, dis, b2.reshape(1, D), W3)
  p3 = _agg(g3, src_a, dst_a, zeros_h)
  return _dfin(p3, g3, dis, b3.reshape(1, D))
